# baseline (speedup 1.0000x reference)
import functools

import jax
import jax.numpy as jnp
from jax import lax
from jax.experimental import pallas as pl
from jax.experimental.pallas import tpu as pltpu
from jax.experimental.pallas import tpu_sc as plsc

B, N, NP, K, CIN, COUT, P = 8, 8192, 2048, 32, 64, 64, 5
RADIUS = 1.5
EPS = 1e-5

NC, NS = 2, 16
NW = NC * NS
NCTR = B * NP
CPW = NCTR // NW
CC = 16
NCHUNK = CPW // CC
RPC = CC * K
GL = RPC // 128
ACC_ROWS = CC * P
NROWS = NCTR


def _prologue_body(lcx_ref, lcy_ref, nl_ref, didx_ref,
                   gidx_ref, sidx_ref, dgidx_ref):
    step = pl.program_id(0)
    blk = lcx_ref.shape
    fi = (lax.broadcasted_iota(jnp.int32, blk, 0) * 128
          + lax.broadcasted_iota(jnp.int32, blk, 1)
          + step * (blk[0] * blk[1]))
    x = lcx_ref[...]
    y = lcy_ref[...]
    dist = jnp.minimum(jnp.sqrt(x * x + y * y) / RADIUS, 0.99)
    bins = jnp.floor(dist * P).astype(jnp.int32)
    c = fi // K
    sid = (c // CPW) % NS
    par = (c // CC) % 4
    sidx_ref[...] = (sid * (4 * ACC_ROWS) + par * ACC_ROWS
                     + (c % CC) * P + bins)
    b = fi // (NP * K)
    gidx_ref[...] = nl_ref[...] + b * N
    dblk = didx_ref.shape
    f2 = (lax.broadcasted_iota(jnp.int32, dblk, 0) * 128
          + lax.broadcasted_iota(jnp.int32, dblk, 1)
          + step * (dblk[0] * dblk[1]))
    dgidx_ref[...] = didx_ref[...] + (f2 // NP) * N


def _sc_body(pts_hbm, gidx_hbm, sidx_hbm, dgidx_hbm, zero_hbm,
             acc_hbm, nxr_hbm,
             gidx_v, sidx_v, rows_v, zeros_v, acc_sh,
             gsem0, gsem1, ssem, zsem0, zsem1, zsem2, zsem3):
    cid = lax.axis_index("c")
    sid = lax.axis_index("s")
    wid = cid * NS + sid
    gsems = (gsem0, gsem1)
    zsems = (zsem0, zsem1, zsem2, zsem3)
    pltpu.sync_copy(zero_hbm, zeros_v)
    irow = wid * (CPW * K // 128)

    def gather_start(p):
        return [
            pltpu.async_copy(pts_hbm.at[gidx_v.at[p * GL + g]],
                             rows_v.at[pl.ds(p * RPC + g * 128, 128)],
                             gsems[p])
            for g in range(GL)
        ]

    def abase_of(r):
        return sid * (4 * ACC_ROWS) + r * ACC_ROWS

    def zero_start(r):
        return pltpu.async_copy(zeros_v, acc_sh.at[pl.ds(abase_of(r), ACC_ROWS)],
                                zsems[r])

    for p in range(2):
        pltpu.sync_copy(gidx_hbm.at[pl.ds(irow + p * GL, GL)],
                        gidx_v.at[pl.ds(p * GL, GL)])
        pltpu.sync_copy(sidx_hbm.at[pl.ds(irow + p * GL, GL)],
                        sidx_v.at[pl.ds(p * GL, GL)])
        gather_start(p)
        zero_start(p)

    def body(i, carry):
        for u in range(4):
            c = 4 * i + u
            p = u % 2
            r = u % 4
            for g in range(GL):
                pltpu.make_async_copy(
                    pts_hbm.at[gidx_v.at[p * GL + g]],
                    rows_v.at[pl.ds(p * RPC + g * 128, 128)],
                    gsems[p]).wait()
            pltpu.make_async_copy(
                zeros_v, acc_sh.at[pl.ds(abase_of(r), ACC_ROWS)],
                zsems[r]).wait()
            descs = [
                pltpu.async_copy(rows_v.at[pl.ds(p * RPC + g * 128, 128)],
                                 acc_sh.at[sidx_v.at[p * GL + g]],
                                 ssem, add=True)
                for g in range(GL)
            ]
            for d in descs:
                d.wait()
            @pl.when(c + 2 < NCHUNK)
            def _():
                pltpu.sync_copy(gidx_hbm.at[pl.ds(irow + (c + 2) * GL, GL)],
                                gidx_v.at[pl.ds(p * GL, GL)])
                pltpu.sync_copy(sidx_hbm.at[pl.ds(irow + (c + 2) * GL, GL)],
                                sidx_v.at[pl.ds(p * GL, GL)])
                gather_start(p)
            cbase = (wid * CPW + c * CC) * P
            pltpu.sync_copy(acc_sh.at[pl.ds(abase_of(r), ACC_ROWS)],
                            acc_hbm.at[pl.ds(cbase, ACC_ROWS)])
            @pl.when(c + 2 < NCHUNK)
            def _():
                zero_start((r + 2) % 4)
        return carry

    lax.fori_loop(0, NCHUNK // 4, body, 0)

    drow = wid * (CPW // 128)
    pltpu.sync_copy(dgidx_hbm.at[pl.ds(drow, GL)],
                    gidx_v.at[pl.ds(0, GL)])
    descs = [
        pltpu.async_copy(pts_hbm.at[gidx_v.at[g]],
                         rows_v.at[pl.ds(g * 128, 128)], gsem0)
        for g in range(GL)
    ]
    for d in descs:
        d.wait()
    pltpu.sync_copy(rows_v.at[pl.ds(0, CPW)], nxr_hbm.at[pl.ds(wid * CPW, CPW)])


def _c1_body(acc_ref, w1_ref, b1_ref, x_ref, st_ref):
    xb = jnp.dot(acc_ref[...], w1_ref[...],
                 preferred_element_type=jnp.float32) + b1_ref[...]
    x_ref[...] = xb
    s = jnp.sum(xb, axis=0, keepdims=True)
    ss = jnp.sum(xb * xb, axis=0, keepdims=True)
    st = jnp.concatenate([s, ss], axis=0)

    @pl.when(pl.program_id(0) == 0)
    def _():
        st_ref[...] = st

    @pl.when(pl.program_id(0) != 0)
    def _():
        st_ref[...] = st_ref[...] + st


def _c2_body(x_ref, st_ref, g1_ref, be1_ref, w2_ref, b2_ref, y_ref, st2_ref):
    st = st_ref[...]
    mu = st[0:1, :] * (1.0 / NROWS)
    var = st[1:2, :] * (1.0 / NROWS) - mu * mu
    xh = (x_ref[...] - mu) / jnp.sqrt(var + EPS) * g1_ref[...] + be1_ref[...]
    xh = jnp.maximum(xh, 0.0)
    yb = jnp.dot(xh, w2_ref[...],
                 preferred_element_type=jnp.float32) + b2_ref[...]
    y_ref[...] = yb
    s = jnp.sum(yb, axis=0, keepdims=True)
    ss = jnp.sum(yb * yb, axis=0, keepdims=True)
    st2 = jnp.concatenate([s, ss], axis=0)

    @pl.when(pl.program_id(0) == 0)
    def _():
        st2_ref[...] = st2

    @pl.when(pl.program_id(0) != 0)
    def _():
        st2_ref[...] = st2_ref[...] + st2


def _c3_body(y_ref, st2_ref, g2_ref, be2_ref, out_ref):
    st = st2_ref[...]
    mu = st[0:1, :] * (1.0 / NROWS)
    var = st[1:2, :] * (1.0 / NROWS) - mu * mu
    yh = (y_ref[...] - mu) / jnp.sqrt(var + EPS) * g2_ref[...] + be2_ref[...]
    out_ref[...] = jnp.maximum(yh, 0.0)


def kernel(xyz, points, local_coordinates, neighbor_lists, parameter_list,
           data_idx, W_conv, b_conv, gamma1, beta1, W_lin, b_lin,
           gamma2, beta2):
    del parameter_list
    pts = jnp.concatenate([points, xyz], axis=2).reshape(B * N, CIN)
    pts = pts.astype(jnp.bfloat16)
    lcx = local_coordinates[..., 0].reshape(-1, 128)
    lcy = local_coordinates[..., 1].reshape(-1, 128)
    nl = neighbor_lists.reshape(-1, 128)
    didx = data_idx.reshape(-1, 128)
    nrows = nl.shape[0]
    drows = didx.shape[0]
    grid = 8
    rb = nrows // grid
    db = drows // grid

    gidx, sidx, dgidx = pl.pallas_call(
        _prologue_body,
        grid=(grid,),
        in_specs=[
            pl.BlockSpec((rb, 128), lambda i: (i, 0)),
            pl.BlockSpec((rb, 128), lambda i: (i, 0)),
            pl.BlockSpec((rb, 128), lambda i: (i, 0)),
            pl.BlockSpec((db, 128), lambda i: (i, 0)),
        ],
        out_specs=[
            pl.BlockSpec((rb, 128), lambda i: (i, 0)),
            pl.BlockSpec((rb, 128), lambda i: (i, 0)),
            pl.BlockSpec((db, 128), lambda i: (i, 0)),
        ],
        out_shape=[
            jax.ShapeDtypeStruct((nrows, 128), jnp.int32),
            jax.ShapeDtypeStruct((nrows, 128), jnp.int32),
            jax.ShapeDtypeStruct((drows, 128), jnp.int32),
        ],
    )(lcx, lcy, nl, didx)

    zero_blk = jnp.zeros((ACC_ROWS, CIN), jnp.bfloat16)

    sc_fn = pl.kernel(
        _sc_body,
        out_type=[
            jax.ShapeDtypeStruct((NCTR * P, CIN), jnp.bfloat16),
            jax.ShapeDtypeStruct((NCTR, CIN), jnp.bfloat16),
        ],
        mesh=plsc.VectorSubcoreMesh(core_axis_name="c", subcore_axis_name="s"),
        scratch_types=[
            pltpu.VMEM((2 * GL, 128), jnp.int32),
            pltpu.VMEM((2 * GL, 128), jnp.int32),
            pltpu.VMEM((2 * RPC, CIN), jnp.bfloat16),
            pltpu.VMEM((ACC_ROWS, CIN), jnp.bfloat16),
            pltpu.VMEM_SHARED((NS * 4 * ACC_ROWS, CIN), jnp.bfloat16),
            pltpu.SemaphoreType.DMA,
            pltpu.SemaphoreType.DMA,
            pltpu.SemaphoreType.DMA,
            pltpu.SemaphoreType.DMA,
            pltpu.SemaphoreType.DMA,
            pltpu.SemaphoreType.DMA,
            pltpu.SemaphoreType.DMA,
        ],
        compiler_params=pltpu.CompilerParams(use_tc_tiling_on_sc=False),
    )
    acc, nxr = sc_fn(pts, gidx, sidx, dgidx, zero_blk)

    feat = acc.reshape(NCTR, P * CIN)
    w1 = W_conv.T
    b1 = b_conv.reshape(1, COUT)
    rows_blk = 1024
    g2 = NCTR // rows_blk

    x, st1 = pl.pallas_call(
        _c1_body,
        grid=(g2,),
        in_specs=[
            pl.BlockSpec((rows_blk, P * CIN), lambda i: (i, 0)),
            pl.BlockSpec((P * CIN, COUT), lambda i: (0, 0)),
            pl.BlockSpec((1, COUT), lambda i: (0, 0)),
        ],
        out_specs=[
            pl.BlockSpec((rows_blk, COUT), lambda i: (i, 0)),
            pl.BlockSpec((2, COUT), lambda i: (0, 0)),
        ],
        out_shape=[
            jax.ShapeDtypeStruct((NCTR, COUT), jnp.float32),
            jax.ShapeDtypeStruct((2, COUT), jnp.float32),
        ],
        compiler_params=pltpu.CompilerParams(
            dimension_semantics=("arbitrary",)),
    )(feat, w1, b1)

    y, st2 = pl.pallas_call(
        _c2_body,
        grid=(g2,),
        in_specs=[
            pl.BlockSpec((rows_blk, COUT), lambda i: (i, 0)),
            pl.BlockSpec((2, COUT), lambda i: (0, 0)),
            pl.BlockSpec((1, COUT), lambda i: (0, 0)),
            pl.BlockSpec((1, COUT), lambda i: (0, 0)),
            pl.BlockSpec((COUT, COUT), lambda i: (0, 0)),
            pl.BlockSpec((1, COUT), lambda i: (0, 0)),
        ],
        out_specs=[
            pl.BlockSpec((rows_blk, COUT), lambda i: (i, 0)),
            pl.BlockSpec((2, COUT), lambda i: (0, 0)),
        ],
        out_shape=[
            jax.ShapeDtypeStruct((NCTR, COUT), jnp.float32),
            jax.ShapeDtypeStruct((2, COUT), jnp.float32),
        ],
        compiler_params=pltpu.CompilerParams(
            dimension_semantics=("arbitrary",)),
    )(x, st1, gamma1.reshape(1, COUT), beta1.reshape(1, COUT),
      W_lin.T, b_lin.reshape(1, COUT))

    new_points = pl.pallas_call(
        _c3_body,
        grid=(g2,),
        in_specs=[
            pl.BlockSpec((rows_blk, COUT), lambda i: (i, 0)),
            pl.BlockSpec((2, COUT), lambda i: (0, 0)),
            pl.BlockSpec((1, COUT), lambda i: (0, 0)),
            pl.BlockSpec((1, COUT), lambda i: (0, 0)),
        ],
        out_specs=pl.BlockSpec((rows_blk, COUT), lambda i: (i, 0)),
        out_shape=jax.ShapeDtypeStruct((NCTR, COUT), jnp.float32),
        compiler_params=pltpu.CompilerParams(
            dimension_semantics=("arbitrary",)),
    )(y, st2, gamma2.reshape(1, COUT), beta2.reshape(1, COUT))

    new_xyz = nxr.reshape(B, NP, CIN)[:, :, CIN - 3:].astype(jnp.float32)
    return (new_xyz, new_points.reshape(B, NP, COUT))


# speedup vs baseline: 1.0012x; 1.0012x over previous
import functools

import jax
import jax.numpy as jnp
from jax import lax
from jax.experimental import pallas as pl
from jax.experimental.pallas import tpu as pltpu
from jax.experimental.pallas import tpu_sc as plsc

B, N, NP, K, CIN, COUT, P = 8, 8192, 2048, 32, 64, 64, 5
RADIUS = 1.5
EPS = 1e-5

NC, NS = 2, 16
NW = NC * NS
NCTR = B * NP
CPW = NCTR // NW
CC = 16
NCHUNK = CPW // CC
RPC = CC * K
GL = RPC // 128
ACC_ROWS = CC * P
NROWS = NCTR


def _prologue_body(lcx_ref, lcy_ref, nl_ref, didx_ref,
                   gidx_ref, sidx_ref, dgidx_ref):
    step = pl.program_id(0)
    blk = lcx_ref.shape
    fi = (lax.broadcasted_iota(jnp.int32, blk, 0) * 128
          + lax.broadcasted_iota(jnp.int32, blk, 1)
          + step * (blk[0] * blk[1]))
    x = lcx_ref[...]
    y = lcy_ref[...]
    dist = jnp.minimum(jnp.sqrt(x * x + y * y) / RADIUS, 0.99)
    bins = jnp.floor(dist * P).astype(jnp.int32)
    c = fi // K
    sid = (c // CPW) % NS
    par = (c // CC) % 4
    sidx_ref[...] = (sid * (4 * ACC_ROWS) + par * ACC_ROWS
                     + (c % CC) * P + bins)
    b = fi // (NP * K)
    gidx_ref[...] = (nl_ref[...] + b * N) * 2
    dblk = didx_ref.shape
    f2 = (lax.broadcasted_iota(jnp.int32, dblk, 0) * 128
          + lax.broadcasted_iota(jnp.int32, dblk, 1)
          + step * (dblk[0] * dblk[1]))
    dgidx_ref[...] = (didx_ref[...] + (f2 // NP) * N) * 2


def _sc_body(pts_hbm, gidx_hbm, sidx_hbm, dgidx_hbm, zero_hbm,
             acc_hbm, nxr_hbm,
             gidx_v, sidx_v, rows_v, zeros_v, acc_sh,
             gsem0, gsem1, ssem, zsem0, zsem1, zsem2, zsem3):
    cid = lax.axis_index("c")
    sid = lax.axis_index("s")
    wid = cid * NS + sid
    gsems = (gsem0, gsem1)
    zsems = (zsem0, zsem1, zsem2, zsem3)
    pltpu.sync_copy(zero_hbm, zeros_v)
    irow = wid * (CPW * K // 128)

    def gather_start(p):
        return [
            pltpu.async_copy(pts_hbm.at[gidx_v.at[p * GL + g]],
                             rows_v.at[pl.ds(p * RPC + g * 128, 128)],
                             gsems[p])
            for g in range(GL)
        ]

    def abase_of(r):
        return sid * (4 * ACC_ROWS) + r * ACC_ROWS

    def zero_start(r):
        return pltpu.async_copy(zeros_v, acc_sh.at[pl.ds(abase_of(r), ACC_ROWS)],
                                zsems[r])

    for p in range(2):
        pltpu.sync_copy(gidx_hbm.at[pl.ds(irow + p * GL, GL)],
                        gidx_v.at[pl.ds(p * GL, GL)])
        pltpu.sync_copy(sidx_hbm.at[pl.ds(irow + p * GL, GL)],
                        sidx_v.at[pl.ds(p * GL, GL)])
        gather_start(p)
        zero_start(p)

    def body(i, carry):
        for u in range(4):
            c = 4 * i + u
            p = u % 2
            r = u % 4
            for g in range(GL):
                pltpu.make_async_copy(
                    pts_hbm.at[gidx_v.at[p * GL + g]],
                    rows_v.at[pl.ds(p * RPC + g * 128, 128)],
                    gsems[p]).wait()
            pltpu.make_async_copy(
                zeros_v, acc_sh.at[pl.ds(abase_of(r), ACC_ROWS)],
                zsems[r]).wait()
            descs = [
                pltpu.async_copy(rows_v.at[pl.ds(p * RPC + g * 128, 128)],
                                 acc_sh.at[sidx_v.at[p * GL + g]],
                                 ssem, add=True)
                for g in range(GL)
            ]
            for d in descs:
                d.wait()
            @pl.when(c + 2 < NCHUNK)
            def _():
                pltpu.sync_copy(gidx_hbm.at[pl.ds(irow + (c + 2) * GL, GL)],
                                gidx_v.at[pl.ds(p * GL, GL)])
                pltpu.sync_copy(sidx_hbm.at[pl.ds(irow + (c + 2) * GL, GL)],
                                sidx_v.at[pl.ds(p * GL, GL)])
                gather_start(p)
            cbase = (wid * CPW + c * CC) * P
            pltpu.sync_copy(acc_sh.at[pl.ds(abase_of(r), ACC_ROWS)],
                            acc_hbm.at[pl.ds(cbase, ACC_ROWS)])
            @pl.when(c + 2 < NCHUNK)
            def _():
                zero_start((r + 2) % 4)
        return carry

    lax.fori_loop(0, NCHUNK // 4, body, 0)

    drow = wid * (CPW // 128)
    pltpu.sync_copy(dgidx_hbm.at[pl.ds(drow, GL)],
                    gidx_v.at[pl.ds(0, GL)])
    descs = [
        pltpu.async_copy(pts_hbm.at[gidx_v.at[g]],
                         rows_v.at[pl.ds(g * 128, 128)], gsem0)
        for g in range(GL)
    ]
    for d in descs:
        d.wait()
    pltpu.sync_copy(rows_v.at[pl.ds(0, CPW)], nxr_hbm.at[pl.ds(wid * CPW, CPW)])


def _c1_body(acc_ref, w1_ref, b1_ref, x_ref, st_ref):
    xb = jnp.dot(acc_ref[...], w1_ref[...],
                 preferred_element_type=jnp.float32) + b1_ref[...]
    x_ref[...] = xb
    s = jnp.sum(xb, axis=0, keepdims=True)
    ss = jnp.sum(xb * xb, axis=0, keepdims=True)
    st = jnp.concatenate([s, ss], axis=0)

    @pl.when(pl.program_id(0) == 0)
    def _():
        st_ref[...] = st

    @pl.when(pl.program_id(0) != 0)
    def _():
        st_ref[...] = st_ref[...] + st


def _c2_body(x_ref, st_ref, g1_ref, be1_ref, w2_ref, b2_ref, y_ref, st2_ref):
    st = st_ref[...]
    mu = st[0:1, :] * (1.0 / NROWS)
    var = st[1:2, :] * (1.0 / NROWS) - mu * mu
    xh = (x_ref[...] - mu) / jnp.sqrt(var + EPS) * g1_ref[...] + be1_ref[...]
    xh = jnp.maximum(xh, 0.0)
    yb = jnp.dot(xh, w2_ref[...],
                 preferred_element_type=jnp.float32) + b2_ref[...]
    y_ref[...] = yb
    s = jnp.sum(yb, axis=0, keepdims=True)
    ss = jnp.sum(yb * yb, axis=0, keepdims=True)
    st2 = jnp.concatenate([s, ss], axis=0)

    @pl.when(pl.program_id(0) == 0)
    def _():
        st2_ref[...] = st2

    @pl.when(pl.program_id(0) != 0)
    def _():
        st2_ref[...] = st2_ref[...] + st2


def _c3_body(y_ref, st2_ref, g2_ref, be2_ref, out_ref):
    st = st2_ref[...]
    mu = st[0:1, :] * (1.0 / NROWS)
    var = st[1:2, :] * (1.0 / NROWS) - mu * mu
    yh = (y_ref[...] - mu) / jnp.sqrt(var + EPS) * g2_ref[...] + be2_ref[...]
    out_ref[...] = jnp.maximum(yh, 0.0)


def kernel(xyz, points, local_coordinates, neighbor_lists, parameter_list,
           data_idx, W_conv, b_conv, gamma1, beta1, W_lin, b_lin,
           gamma2, beta2):
    del parameter_list
    pts128 = jnp.concatenate(
        [points.reshape(B * N, CIN - 3), xyz.reshape(B * N, 3),
         jnp.zeros((B * N, CIN), jnp.float32)], axis=1)
    pts = pts128.reshape(2 * B * N, CIN)
    lcx = local_coordinates[..., 0].reshape(-1, 128)
    lcy = local_coordinates[..., 1].reshape(-1, 128)
    nl = neighbor_lists.reshape(-1, 128)
    didx = data_idx.reshape(-1, 128)
    nrows = nl.shape[0]
    drows = didx.shape[0]
    grid = 8
    rb = nrows // grid
    db = drows // grid

    gidx, sidx, dgidx = pl.pallas_call(
        _prologue_body,
        grid=(grid,),
        in_specs=[
            pl.BlockSpec((rb, 128), lambda i: (i, 0)),
            pl.BlockSpec((rb, 128), lambda i: (i, 0)),
            pl.BlockSpec((rb, 128), lambda i: (i, 0)),
            pl.BlockSpec((db, 128), lambda i: (i, 0)),
        ],
        out_specs=[
            pl.BlockSpec((rb, 128), lambda i: (i, 0)),
            pl.BlockSpec((rb, 128), lambda i: (i, 0)),
            pl.BlockSpec((db, 128), lambda i: (i, 0)),
        ],
        out_shape=[
            jax.ShapeDtypeStruct((nrows, 128), jnp.int32),
            jax.ShapeDtypeStruct((nrows, 128), jnp.int32),
            jax.ShapeDtypeStruct((drows, 128), jnp.int32),
        ],
    )(lcx, lcy, nl, didx)

    zero_blk = jnp.zeros((ACC_ROWS, CIN), jnp.float32)

    sc_fn = pl.kernel(
        _sc_body,
        out_type=[
            jax.ShapeDtypeStruct((NCTR * P, CIN), jnp.float32),
            jax.ShapeDtypeStruct((NCTR, CIN), jnp.float32),
        ],
        mesh=plsc.VectorSubcoreMesh(core_axis_name="c", subcore_axis_name="s"),
        scratch_types=[
            pltpu.VMEM((2 * GL, 128), jnp.int32),
            pltpu.VMEM((2 * GL, 128), jnp.int32),
            pltpu.VMEM((2 * RPC, CIN), jnp.float32),
            pltpu.VMEM((ACC_ROWS, CIN), jnp.float32),
            pltpu.VMEM_SHARED((NS * 4 * ACC_ROWS, CIN), jnp.float32),
            pltpu.SemaphoreType.DMA,
            pltpu.SemaphoreType.DMA,
            pltpu.SemaphoreType.DMA,
            pltpu.SemaphoreType.DMA,
            pltpu.SemaphoreType.DMA,
            pltpu.SemaphoreType.DMA,
            pltpu.SemaphoreType.DMA,
        ],
        compiler_params=pltpu.CompilerParams(use_tc_tiling_on_sc=False),
    )
    acc, nxr = sc_fn(pts, gidx, sidx, dgidx, zero_blk)

    feat = acc.reshape(NCTR, P * CIN)
    w1 = W_conv.T
    b1 = b_conv.reshape(1, COUT)
    rows_blk = 1024
    g2 = NCTR // rows_blk

    x, st1 = pl.pallas_call(
        _c1_body,
        grid=(g2,),
        in_specs=[
            pl.BlockSpec((rows_blk, P * CIN), lambda i: (i, 0)),
            pl.BlockSpec((P * CIN, COUT), lambda i: (0, 0)),
            pl.BlockSpec((1, COUT), lambda i: (0, 0)),
        ],
        out_specs=[
            pl.BlockSpec((rows_blk, COUT), lambda i: (i, 0)),
            pl.BlockSpec((2, COUT), lambda i: (0, 0)),
        ],
        out_shape=[
            jax.ShapeDtypeStruct((NCTR, COUT), jnp.float32),
            jax.ShapeDtypeStruct((2, COUT), jnp.float32),
        ],
        compiler_params=pltpu.CompilerParams(
            dimension_semantics=("arbitrary",)),
    )(feat, w1, b1)

    y, st2 = pl.pallas_call(
        _c2_body,
        grid=(g2,),
        in_specs=[
            pl.BlockSpec((rows_blk, COUT), lambda i: (i, 0)),
            pl.BlockSpec((2, COUT), lambda i: (0, 0)),
            pl.BlockSpec((1, COUT), lambda i: (0, 0)),
            pl.BlockSpec((1, COUT), lambda i: (0, 0)),
            pl.BlockSpec((COUT, COUT), lambda i: (0, 0)),
            pl.BlockSpec((1, COUT), lambda i: (0, 0)),
        ],
        out_specs=[
            pl.BlockSpec((rows_blk, COUT), lambda i: (i, 0)),
            pl.BlockSpec((2, COUT), lambda i: (0, 0)),
        ],
        out_shape=[
            jax.ShapeDtypeStruct((NCTR, COUT), jnp.float32),
            jax.ShapeDtypeStruct((2, COUT), jnp.float32),
        ],
        compiler_params=pltpu.CompilerParams(
            dimension_semantics=("arbitrary",)),
    )(x, st1, gamma1.reshape(1, COUT), beta1.reshape(1, COUT),
      W_lin.T, b_lin.reshape(1, COUT))

    new_points = pl.pallas_call(
        _c3_body,
        grid=(g2,),
        in_specs=[
            pl.BlockSpec((rows_blk, COUT), lambda i: (i, 0)),
            pl.BlockSpec((2, COUT), lambda i: (0, 0)),
            pl.BlockSpec((1, COUT), lambda i: (0, 0)),
            pl.BlockSpec((1, COUT), lambda i: (0, 0)),
        ],
        out_specs=pl.BlockSpec((rows_blk, COUT), lambda i: (i, 0)),
        out_shape=jax.ShapeDtypeStruct((NCTR, COUT), jnp.float32),
        compiler_params=pltpu.CompilerParams(
            dimension_semantics=("arbitrary",)),
    )(y, st2, gamma2.reshape(1, COUT), beta2.reshape(1, COUT))

    new_xyz = nxr.reshape(B, NP, CIN)[:, :, CIN - 3:]
    return (new_xyz, new_points.reshape(B, NP, COUT))


# speedup vs baseline: 1.0640x; 1.0627x over previous
import functools

import jax
import jax.numpy as jnp
from jax import lax
from jax.experimental import pallas as pl
from jax.experimental.pallas import tpu as pltpu
from jax.experimental.pallas import tpu_sc as plsc

B, N, NP, K, CIN, COUT, P = 8, 8192, 2048, 32, 64, 64, 5
RADIUS = 1.5
EPS = 1e-5

NC, NS = 2, 16
NW = NC * NS
NCTR = B * NP
CPW = NCTR // NW
CC = 16
NCHUNK = CPW // CC
RPC = CC * K
GL = RPC // 128
ACC_ROWS = CC * P
NROWS = NCTR


def _prologue_body(lcx_ref, lcy_ref, nl_ref, didx_ref,
                   gidx_ref, sidx_ref, dgidx_ref):
    step = pl.program_id(0)
    blk = lcx_ref.shape
    fi = (lax.broadcasted_iota(jnp.int32, blk, 0) * 128
          + lax.broadcasted_iota(jnp.int32, blk, 1)
          + step * (blk[0] * blk[1]))
    x = lcx_ref[...]
    y = lcy_ref[...]
    dist = jnp.minimum(jnp.sqrt(x * x + y * y) / RADIUS, 0.99)
    bins = jnp.floor(dist * P).astype(jnp.int32)
    c = fi // K
    sid = (c // CPW) % NS
    par = (c // CC) % 4
    sidx_ref[...] = (sid * (4 * ACC_ROWS) + par * ACC_ROWS
                     + (c % CC) * P + bins)
    b = fi // (NP * K)
    gidx_ref[...] = nl_ref[...] + b * N
    dblk = didx_ref.shape
    f2 = (lax.broadcasted_iota(jnp.int32, dblk, 0) * 128
          + lax.broadcasted_iota(jnp.int32, dblk, 1)
          + step * (dblk[0] * dblk[1]))
    dgidx_ref[...] = didx_ref[...] + (f2 // NP) * N


def _sc_body(pts_hbm, iidx_hbm, dgidx_hbm, zero_hbm,
             acc_hbm, nxr_hbm,
             cidx_v, rows_v, zeros_v, acc_sh,
             gsem0, gsem1, ssem, zsem0, zsem1, zsem2, zsem3):
    cid = lax.axis_index("c")
    sid = lax.axis_index("s")
    wid = cid * NS + sid
    gsems = (gsem0, gsem1)
    zsems = (zsem0, zsem1, zsem2, zsem3)
    pltpu.sync_copy(zero_hbm, zeros_v)
    irow = wid * (NCHUNK * 2 * GL)

    def idx_load(c, p):
        pltpu.sync_copy(iidx_hbm.at[pl.ds(irow + c * 2 * GL, 2 * GL)],
                        cidx_v.at[pl.ds(p * 2 * GL, 2 * GL)])

    def gather_start(p):
        return [
            pltpu.async_copy(pts_hbm.at[cidx_v.at[p * 2 * GL + g]],
                             rows_v.at[pl.ds(p * RPC + g * 128, 128)],
                             gsems[p])
            for g in range(GL)
        ]

    def abase_of(r):
        return sid * (4 * ACC_ROWS) + r * ACC_ROWS

    def zero_start(r):
        return pltpu.async_copy(zeros_v, acc_sh.at[pl.ds(abase_of(r), ACC_ROWS)],
                                zsems[r])

    for p in range(2):
        idx_load(p, p)
        gather_start(p)
        zero_start(p)

    def body(i, carry):
        for u in range(4):
            c = 4 * i + u
            p = u % 2
            r = u % 4
            for g in range(GL):
                pltpu.make_async_copy(
                    pts_hbm.at[cidx_v.at[p * 2 * GL + g]],
                    rows_v.at[pl.ds(p * RPC + g * 128, 128)],
                    gsems[p]).wait()
            pltpu.make_async_copy(
                zeros_v, acc_sh.at[pl.ds(abase_of(r), ACC_ROWS)],
                zsems[r]).wait()
            descs = [
                pltpu.async_copy(rows_v.at[pl.ds(p * RPC + g * 128, 128)],
                                 acc_sh.at[cidx_v.at[p * 2 * GL + GL + g]],
                                 ssem, add=True)
                for g in range(GL)
            ]
            for d in descs:
                d.wait()
            @pl.when(c + 2 < NCHUNK)
            def _():
                idx_load(c + 2, p)
                gather_start(p)
            cbase = (wid * CPW + c * CC) * P
            pltpu.sync_copy(acc_sh.at[pl.ds(abase_of(r), ACC_ROWS)],
                            acc_hbm.at[pl.ds(cbase, ACC_ROWS)])
            @pl.when(c + 2 < NCHUNK)
            def _():
                zero_start((r + 2) % 4)
        return carry

    lax.fori_loop(0, NCHUNK // 4, body, 0)

    drow = wid * (CPW // 128)
    pltpu.sync_copy(dgidx_hbm.at[pl.ds(drow, GL)],
                    cidx_v.at[pl.ds(0, GL)])
    descs = [
        pltpu.async_copy(pts_hbm.at[cidx_v.at[g]],
                         rows_v.at[pl.ds(g * 128, 128)], gsem0)
        for g in range(GL)
    ]
    for d in descs:
        d.wait()
    pltpu.sync_copy(rows_v.at[pl.ds(0, CPW)], nxr_hbm.at[pl.ds(wid * CPW, CPW)])


def _c1_body(acc_ref, w1_ref, b1_ref, x_ref, st_ref):
    xb = jnp.dot(acc_ref[...], w1_ref[...],
                 preferred_element_type=jnp.float32) + b1_ref[...]
    x_ref[...] = xb
    s = jnp.sum(xb, axis=0, keepdims=True)
    ss = jnp.sum(xb * xb, axis=0, keepdims=True)
    st = jnp.concatenate([s, ss], axis=0)

    @pl.when(pl.program_id(0) == 0)
    def _():
        st_ref[...] = st

    @pl.when(pl.program_id(0) != 0)
    def _():
        st_ref[...] = st_ref[...] + st


def _c2_body(x_ref, st_ref, g1_ref, be1_ref, w2_ref, b2_ref, y_ref, st2_ref):
    st = st_ref[...]
    mu = st[0:1, :] * (1.0 / NROWS)
    var = st[1:2, :] * (1.0 / NROWS) - mu * mu
    xh = (x_ref[...] - mu) / jnp.sqrt(var + EPS) * g1_ref[...] + be1_ref[...]
    xh = jnp.maximum(xh, 0.0)
    yb = jnp.dot(xh, w2_ref[...],
                 preferred_element_type=jnp.float32) + b2_ref[...]
    y_ref[...] = yb
    s = jnp.sum(yb, axis=0, keepdims=True)
    ss = jnp.sum(yb * yb, axis=0, keepdims=True)
    st2 = jnp.concatenate([s, ss], axis=0)

    @pl.when(pl.program_id(0) == 0)
    def _():
        st2_ref[...] = st2

    @pl.when(pl.program_id(0) != 0)
    def _():
        st2_ref[...] = st2_ref[...] + st2


def _c3_body(y_ref, st2_ref, g2_ref, be2_ref, out_ref):
    st = st2_ref[...]
    mu = st[0:1, :] * (1.0 / NROWS)
    var = st[1:2, :] * (1.0 / NROWS) - mu * mu
    yh = (y_ref[...] - mu) / jnp.sqrt(var + EPS) * g2_ref[...] + be2_ref[...]
    out_ref[...] = jnp.maximum(yh, 0.0)


def kernel(xyz, points, local_coordinates, neighbor_lists, parameter_list,
           data_idx, W_conv, b_conv, gamma1, beta1, W_lin, b_lin,
           gamma2, beta2):
    del parameter_list
    pts = jnp.concatenate([points, xyz], axis=2).reshape(B * N, CIN)
    lcx = local_coordinates[..., 0].reshape(-1, 128)
    lcy = local_coordinates[..., 1].reshape(-1, 128)
    nl = neighbor_lists.reshape(-1, 128)
    didx = data_idx.reshape(-1, 128)
    nrows = nl.shape[0]
    drows = didx.shape[0]
    grid = 8
    rb = nrows // grid
    db = drows // grid

    gidx, sidx, dgidx = pl.pallas_call(
        _prologue_body,
        grid=(grid,),
        in_specs=[
            pl.BlockSpec((rb, 128), lambda i: (i, 0)),
            pl.BlockSpec((rb, 128), lambda i: (i, 0)),
            pl.BlockSpec((rb, 128), lambda i: (i, 0)),
            pl.BlockSpec((db, 128), lambda i: (i, 0)),
        ],
        out_specs=[
            pl.BlockSpec((rb, 128), lambda i: (i, 0)),
            pl.BlockSpec((rb, 128), lambda i: (i, 0)),
            pl.BlockSpec((db, 128), lambda i: (i, 0)),
        ],
        out_shape=[
            jax.ShapeDtypeStruct((nrows, 128), jnp.int32),
            jax.ShapeDtypeStruct((nrows, 128), jnp.int32),
            jax.ShapeDtypeStruct((drows, 128), jnp.int32),
        ],
    )(lcx, lcy, nl, didx)

    zero_blk = jnp.zeros((ACC_ROWS, CIN), jnp.float32)

    sc_fn = pl.kernel(
        _sc_body,
        out_type=[
            jax.ShapeDtypeStruct((NCTR * P, CIN), jnp.float32),
            jax.ShapeDtypeStruct((NCTR, CIN), jnp.float32),
        ],
        mesh=plsc.VectorSubcoreMesh(core_axis_name="c", subcore_axis_name="s"),
        scratch_types=[
            pltpu.VMEM((2 * 2 * GL, 128), jnp.int32),
            pltpu.VMEM((2 * RPC, CIN), jnp.float32),
            pltpu.VMEM((ACC_ROWS, CIN), jnp.float32),
            pltpu.VMEM_SHARED((NS * 4 * ACC_ROWS, CIN), jnp.float32),
            pltpu.SemaphoreType.DMA,
            pltpu.SemaphoreType.DMA,
            pltpu.SemaphoreType.DMA,
            pltpu.SemaphoreType.DMA,
            pltpu.SemaphoreType.DMA,
            pltpu.SemaphoreType.DMA,
            pltpu.SemaphoreType.DMA,
        ],
        compiler_params=pltpu.CompilerParams(use_tc_tiling_on_sc=False),
    )
    iidx = jnp.concatenate(
        [gidx.reshape(NW, NCHUNK, GL, 128),
         sidx.reshape(NW, NCHUNK, GL, 128)], axis=2).reshape(-1, 128)
    acc, nxr = sc_fn(pts, iidx, dgidx, zero_blk)

    feat = acc.reshape(NCTR, P * CIN)
    w1 = W_conv.T
    b1 = b_conv.reshape(1, COUT)
    rows_blk = 1024
    g2 = NCTR // rows_blk

    x, st1 = pl.pallas_call(
        _c1_body,
        grid=(g2,),
        in_specs=[
            pl.BlockSpec((rows_blk, P * CIN), lambda i: (i, 0)),
            pl.BlockSpec((P * CIN, COUT), lambda i: (0, 0)),
            pl.BlockSpec((1, COUT), lambda i: (0, 0)),
        ],
        out_specs=[
            pl.BlockSpec((rows_blk, COUT), lambda i: (i, 0)),
            pl.BlockSpec((2, COUT), lambda i: (0, 0)),
        ],
        out_shape=[
            jax.ShapeDtypeStruct((NCTR, COUT), jnp.float32),
            jax.ShapeDtypeStruct((2, COUT), jnp.float32),
        ],
        compiler_params=pltpu.CompilerParams(
            dimension_semantics=("arbitrary",)),
    )(feat, w1, b1)

    y, st2 = pl.pallas_call(
        _c2_body,
        grid=(g2,),
        in_specs=[
            pl.BlockSpec((rows_blk, COUT), lambda i: (i, 0)),
            pl.BlockSpec((2, COUT), lambda i: (0, 0)),
            pl.BlockSpec((1, COUT), lambda i: (0, 0)),
            pl.BlockSpec((1, COUT), lambda i: (0, 0)),
            pl.BlockSpec((COUT, COUT), lambda i: (0, 0)),
            pl.BlockSpec((1, COUT), lambda i: (0, 0)),
        ],
        out_specs=[
            pl.BlockSpec((rows_blk, COUT), lambda i: (i, 0)),
            pl.BlockSpec((2, COUT), lambda i: (0, 0)),
        ],
        out_shape=[
            jax.ShapeDtypeStruct((NCTR, COUT), jnp.float32),
            jax.ShapeDtypeStruct((2, COUT), jnp.float32),
        ],
        compiler_params=pltpu.CompilerParams(
            dimension_semantics=("arbitrary",)),
    )(x, st1, gamma1.reshape(1, COUT), beta1.reshape(1, COUT),
      W_lin.T, b_lin.reshape(1, COUT))

    new_points = pl.pallas_call(
        _c3_body,
        grid=(g2,),
        in_specs=[
            pl.BlockSpec((rows_blk, COUT), lambda i: (i, 0)),
            pl.BlockSpec((2, COUT), lambda i: (0, 0)),
            pl.BlockSpec((1, COUT), lambda i: (0, 0)),
            pl.BlockSpec((1, COUT), lambda i: (0, 0)),
        ],
        out_specs=pl.BlockSpec((rows_blk, COUT), lambda i: (i, 0)),
        out_shape=jax.ShapeDtypeStruct((NCTR, COUT), jnp.float32),
        compiler_params=pltpu.CompilerParams(
            dimension_semantics=("arbitrary",)),
    )(y, st2, gamma2.reshape(1, COUT), beta2.reshape(1, COUT))

    new_xyz = nxr.reshape(B, NP, CIN)[:, :, CIN - 3:]
    return (new_xyz, new_points.reshape(B, NP, COUT))
